# baseline (speedup 1.0000x reference)
import functools

import jax
import jax.numpy as jnp
from jax import lax
from jax.experimental import pallas as pl
from jax.experimental.pallas import tpu as pltpu
from jax.experimental.pallas import tpu_sc as plsc

_N = 10000
_E = 320000
_D = 128
_H = 16

_NC = 2
_NS = 16
_NW = _NC * _NS
_EPW = _E // _NW
_SUB = 80
_NROW = _EPW // _SUB
_GRP = 5

_PREC = jax.lax.Precision.HIGHEST
_F32 = jnp.float32

_vector_mesh = plsc.VectorSubcoreMesh(core_axis_name="c", subcore_axis_name="s")
_SC_PARAMS = pltpu.CompilerParams(
    use_tc_tiling_on_sc=False, needs_layout_passes=False)


def _dot(a, b):
    return jnp.dot(a, b, preferred_element_type=_F32, precision=_PREC)


def _node_proj_body(x_ref, wa_ref, wb_ref, xa_ref, xb_ref):
    xv = x_ref[...]
    xa_ref[...] = _dot(xv, wa_ref[...])
    xb_ref[...] = _dot(xv, wb_ref[...])


def _node_proj(x8, wa_blk, wb_blk):
    return pl.pallas_call(
        _node_proj_body,
        out_shape=[
            jax.ShapeDtypeStruct((_N // 8, 128), _F32),
            jax.ShapeDtypeStruct((_N // 8, 128), _F32),
        ],
    )(x8, wa_blk, wb_blk)


def _sc_gather_body(xa_hbm, xb_hbm, src_hbm, dst_hbm, ga_hbm, gb_hbm,
                    srcv, dstv, ga0, gb0, ga1, gb1, gsem0, gsem1, wsem):
    wid = lax.axis_index("s") * _NC + lax.axis_index("c")
    pltpu.sync_copy(src_hbm.at[wid], srcv)
    pltpu.sync_copy(dst_hbm.at[wid], dstv)

    def fire(j0, ga, gb, gsem):
        ds = []
        for k in range(_GRP):
            sl = pl.ds(k * _SUB, _SUB)
            ds.append(pltpu.async_copy(xa_hbm.at[srcv.at[j0 + k]], ga.at[sl], gsem))
            ds.append(pltpu.async_copy(xb_hbm.at[dstv.at[j0 + k]], gb.at[sl], gsem))
        return ds

    @pl.loop(0, _NROW - _GRP, step=2 * _GRP)
    def _(j):
        g0 = fire(j, ga0, gb0, gsem0)
        g1 = fire(j + _GRP, ga1, gb1, gsem1)
        base0 = wid * _EPW + j * _SUB
        for d in g0:
            d.wait()
        wa = pltpu.async_copy(ga0, ga_hbm.at[pl.ds(base0, _GRP * _SUB)], wsem)
        wb = pltpu.async_copy(gb0, gb_hbm.at[pl.ds(base0, _GRP * _SUB)], wsem)
        for d in g1:
            d.wait()
        base1 = base0 + _GRP * _SUB
        wc = pltpu.async_copy(ga1, ga_hbm.at[pl.ds(base1, _GRP * _SUB)], wsem)
        wd = pltpu.async_copy(gb1, gb_hbm.at[pl.ds(base1, _GRP * _SUB)], wsem)
        for d in (wa, wb, wc, wd):
            d.wait()

    jt = _NROW - _GRP
    gt = fire(jt, ga0, gb0, gsem0)
    for d in gt:
        d.wait()
    baset = wid * _EPW + jt * _SUB
    pltpu.sync_copy(ga0, ga_hbm.at[pl.ds(baset, _GRP * _SUB)])
    pltpu.sync_copy(gb0, gb_hbm.at[pl.ds(baset, _GRP * _SUB)])


def _sc_gather(xa, xb, src3, dst3):
    k = pl.kernel(
        _sc_gather_body,
        out_type=[
            jax.ShapeDtypeStruct((_E, _H), _F32),
            jax.ShapeDtypeStruct((_E, _H), _F32),
        ],
        mesh=_vector_mesh,
        scratch_types=[
            pltpu.VMEM((_NROW, _SUB), jnp.int32),
            pltpu.VMEM((_NROW, _SUB), jnp.int32),
            pltpu.VMEM((_GRP * _SUB, _H), _F32),
            pltpu.VMEM((_GRP * _SUB, _H), _F32),
            pltpu.VMEM((_GRP * _SUB, _H), _F32),
            pltpu.VMEM((_GRP * _SUB, _H), _F32),
            pltpu.SemaphoreType.DMA,
            pltpu.SemaphoreType.DMA,
            pltpu.SemaphoreType.DMA,
        ],
        compiler_params=_SC_PARAMS,
    )
    return k(xa, xb, src3, dst3)


def _mlp_body(ga_ref, gb_ref, ea_ref, w1c_ref, b1_ref, w2_ref, b2_ref,
              w3_ref, b3_ref, h3_ref):
    g = (ga_ref[...] + gb_ref[...] + b1_ref[...]
         + _dot(ea_ref[...], w1c_ref[...]))
    g = jnp.maximum(g, 0.0)
    h = jnp.maximum(_dot(g, w2_ref[...]) + b2_ref[...], 0.0)
    h = jnp.maximum(_dot(h, w3_ref[...]) + b3_ref[...], 0.0)
    h3_ref[...] = h


def _mlp(ga8, gb8, ea32, w1c_blk, b1t, w2_blk, b2t, w3_blk, b3t):
    rows = _E // 8
    rb = rows // 10
    full = lambda shape: pl.BlockSpec(shape, lambda i: (0, 0))
    return pl.pallas_call(
        _mlp_body,
        grid=(10,),
        in_specs=[
            pl.BlockSpec((rb, 128), lambda i: (i, 0)),
            pl.BlockSpec((rb, 128), lambda i: (i, 0)),
            pl.BlockSpec((rb, 32), lambda i: (i, 0)),
            full((32, 128)),
            full((1, 128)),
            full((128, 128)),
            full((1, 128)),
            full((128, 128)),
            full((1, 128)),
        ],
        out_specs=pl.BlockSpec((rb, 128), lambda i: (i, 0)),
        out_shape=jax.ShapeDtypeStruct((rows, 128), _F32),
    )(ga8, gb8, ea32, w1c_blk, b1t, w2_blk, b2t, w3_blk, b3t)


def _sc_scatter_body(h3_hbm, dsti_hbm, zero_hbm, s_hbm, cnt_hbm,
                     dstv, h0, h1, cntv, s_sh,
                     lsem0, lsem1, asem0, asem1):
    cid = lax.axis_index("c")
    sid = lax.axis_index("s")
    wid = sid * _NC + cid
    z16 = jnp.zeros((16,), _F32)
    ones16 = jnp.ones((16,), _F32)

    @pl.when(sid == 0)
    def _():
        pltpu.sync_copy(zero_hbm, s_sh)

    pltpu.sync_copy(dsti_hbm.at[wid], dstv)

    @pl.loop(0, _N, step=64)
    def _(i):
        for u in range(4):
            cntv[pl.ds(i + u * 16, 16)] = z16

    plsc.subcore_barrier()

    def fire_add(j0, h, asem):
        ds = []
        for k in range(_GRP):
            sl = pl.ds(k * _SUB, _SUB)
            ds.append(pltpu.async_copy(
                h.at[sl], s_sh.at[dstv.at[j0 + k]], asem, add=True))
        return ds

    @pl.loop(0, _NROW - _GRP, step=2 * _GRP)
    def _(j):
        base0 = wid * _EPW + j * _SUB
        l0 = pltpu.async_copy(
            h3_hbm.at[pl.ds(base0, _GRP * _SUB)], h0, lsem0)
        l1 = pltpu.async_copy(
            h3_hbm.at[pl.ds(base0 + _GRP * _SUB, _GRP * _SUB)], h1, lsem1)
        l0.wait()
        a0 = fire_add(j, h0, asem0)
        l1.wait()
        a1 = fire_add(j + _GRP, h1, asem1)
        for d in a0 + a1:
            d.wait()

    jt = _NROW - _GRP
    pltpu.sync_copy(h3_hbm.at[pl.ds(wid * _EPW + jt * _SUB, _GRP * _SUB)], h0)
    for d in fire_add(jt, h0, asem0):
        d.wait()

    @pl.loop(0, _NROW)
    def _(j):
        for u in range(_SUB // 16):
            plsc.addupdate_scatter(cntv, [dstv[j, pl.ds(u * 16, 16)]], ones16)

    plsc.subcore_barrier()

    @pl.when(sid == 0)
    def _():
        pltpu.sync_copy(s_sh, s_hbm.at[cid])

    pltpu.sync_copy(cntv, cnt_hbm.at[wid])


def _sc_scatter(h3, dst3, zeros):
    k = pl.kernel(
        _sc_scatter_body,
        out_type=[
            jax.ShapeDtypeStruct((_NC, _N, _H), _F32),
            jax.ShapeDtypeStruct((_NW, _N), _F32),
        ],
        mesh=_vector_mesh,
        scratch_types=[
            pltpu.VMEM((_NROW, _SUB), jnp.int32),
            pltpu.VMEM((_GRP * _SUB, _H), _F32),
            pltpu.VMEM((_GRP * _SUB, _H), _F32),
            pltpu.VMEM((_N,), _F32),
            pltpu.VMEM_SHARED((_N, _H), _F32),
            pltpu.SemaphoreType.DMA,
            pltpu.SemaphoreType.DMA,
            pltpu.SemaphoreType.DMA,
            pltpu.SemaphoreType.DMA,
        ],
        compiler_params=_SC_PARAMS,
    )
    return k(h3, dst3, zeros)


def _out_body(s_ref, c_ref, w4_ref, b4_ref, o_ref):
    s = s_ref[0] + s_ref[1]
    c = jnp.sum(c_ref[...].reshape(_NW, _N // 16, 16), axis=0)
    o_ref[...] = _dot(s, w4_ref[...]) + _dot(c, b4_ref[...])


def _out_gemm(s625, c3, w4_blk, b4_blk):
    return pl.pallas_call(
        _out_body,
        out_shape=jax.ShapeDtypeStruct((_N // 16, 16 * _D), _F32),
    )(s625, c3, w4_blk, b4_blk)


def kernel(x, edge_index, edge_attr, W1, b1, W2, b2, W3, b3, W4, b4):
    w1a = W1[:_D]
    w1b = W1[_D:2 * _D]
    w1c = W1[2 * _D:]

    eye8 = jnp.eye(8, dtype=_F32)
    wa_blk = jnp.kron(eye8, w1a)
    wb_blk = jnp.kron(eye8, w1b)
    w1c_blk = jnp.kron(eye8, w1c)
    w2_blk = jnp.kron(eye8, W2)
    w3_blk = jnp.kron(eye8, W3)
    b1t = jnp.tile(b1, 8).reshape(1, 128)
    b2t = jnp.tile(b2, 8).reshape(1, 128)
    b3t = jnp.tile(b3, 8).reshape(1, 128)
    eye16 = jnp.eye(16, dtype=_F32)
    w4_blk = jnp.kron(eye16, W4)
    b4_blk = jnp.kron(eye16, b4.reshape(1, _D))

    src3 = edge_index[0].reshape(_NW, _NROW, _SUB)
    dst3 = edge_index[1].reshape(_NW, _NROW, _SUB)

    xa8, xb8 = _node_proj(x.reshape(_N // 8, 8 * _D), wa_blk, wb_blk)
    ga, gb = _sc_gather(
        xa8.reshape(_N, _H), xb8.reshape(_N, _H), src3, dst3)

    ea_t = jax.lax.optimization_barrier(edge_attr.T)
    ea128 = ea_t.reshape(4, _N, 4, 8).transpose(1, 2, 3, 0).reshape(_N, 128)
    ea32 = ea128.reshape(_E // 8, 32)

    h3 = _mlp(
        ga.reshape(_E // 8, 128),
        gb.reshape(_E // 8, 128),
        ea32,
        w1c_blk, b1t, w2_blk, b2t, w3_blk, b3t,
    )

    zeros = jnp.zeros((_N, _H), _F32)
    s_part, cnt = _sc_scatter(h3.reshape(_E, _H), dst3, zeros)

    out = _out_gemm(
        s_part.reshape(_NC, _N // 16, 16 * _H),
        cnt.reshape(_NW * (_N // 16), 16),
        w4_blk, b4_blk,
    )
    return out.reshape(_N, _D)


# speedup vs baseline: 1.9701x; 1.9701x over previous
import functools

import jax
import jax.numpy as jnp
from jax import lax
from jax.experimental import pallas as pl
from jax.experimental.pallas import tpu as pltpu
from jax.experimental.pallas import tpu_sc as plsc

_N = 10000
_E = 320000
_D = 128
_H = 16

_NC = 2
_NS = 16
_NW = _NC * _NS
_EPW = _E // _NW
_SUB = 80
_NROW = _EPW // _SUB
_GRP = 5

_PREC = jax.lax.Precision.HIGHEST
_F32 = jnp.float32

_vector_mesh = plsc.VectorSubcoreMesh(core_axis_name="c", subcore_axis_name="s")
_SC_PARAMS = pltpu.CompilerParams(
    use_tc_tiling_on_sc=False, needs_layout_passes=False)


def _dot(a, b):
    return jnp.dot(a, b, preferred_element_type=_F32, precision=_PREC)


def _node_proj_body(x_ref, wa_ref, wb_ref, xa_ref, xb_ref):
    xv = x_ref[...]
    xa_ref[...] = _dot(xv, wa_ref[...])
    xb_ref[...] = _dot(xv, wb_ref[...])


def _node_proj(x8, wa_blk, wb_blk):
    return pl.pallas_call(
        _node_proj_body,
        out_shape=[
            jax.ShapeDtypeStruct((_N // 8, 128), _F32),
            jax.ShapeDtypeStruct((_N // 8, 128), _F32),
        ],
    )(x8, wa_blk, wb_blk)


def _sc_gather_body(xa_hbm, xb_hbm, src_hbm, dst_hbm, ga_hbm, gb_hbm,
                    srcv, dstv, ga0, gb0, ga1, gb1, gsem0, gsem1, wsem):
    wid = lax.axis_index("s") * _NC + lax.axis_index("c")
    pltpu.sync_copy(src_hbm.at[wid], srcv)
    pltpu.sync_copy(dst_hbm.at[wid], dstv)

    def fire(j0, ga, gb, gsem):
        ds = []
        for k in range(_GRP):
            sl = pl.ds(k * _SUB, _SUB)
            ds.append(pltpu.async_copy(xa_hbm.at[srcv.at[j0 + k]], ga.at[sl], gsem))
            ds.append(pltpu.async_copy(xb_hbm.at[dstv.at[j0 + k]], gb.at[sl], gsem))
        return ds

    @pl.loop(0, _NROW - _GRP, step=2 * _GRP)
    def _(j):
        g0 = fire(j, ga0, gb0, gsem0)
        g1 = fire(j + _GRP, ga1, gb1, gsem1)
        base0 = wid * _EPW + j * _SUB
        for d in g0:
            d.wait()
        wa = pltpu.async_copy(ga0, ga_hbm.at[pl.ds(base0, _GRP * _SUB)], wsem)
        wb = pltpu.async_copy(gb0, gb_hbm.at[pl.ds(base0, _GRP * _SUB)], wsem)
        for d in g1:
            d.wait()
        base1 = base0 + _GRP * _SUB
        wc = pltpu.async_copy(ga1, ga_hbm.at[pl.ds(base1, _GRP * _SUB)], wsem)
        wd = pltpu.async_copy(gb1, gb_hbm.at[pl.ds(base1, _GRP * _SUB)], wsem)
        for d in (wa, wb, wc, wd):
            d.wait()

    jt = _NROW - _GRP
    gt = fire(jt, ga0, gb0, gsem0)
    for d in gt:
        d.wait()
    baset = wid * _EPW + jt * _SUB
    pltpu.sync_copy(ga0, ga_hbm.at[pl.ds(baset, _GRP * _SUB)])
    pltpu.sync_copy(gb0, gb_hbm.at[pl.ds(baset, _GRP * _SUB)])


def _sc_gather(xa, xb, src3, dst3):
    k = pl.kernel(
        _sc_gather_body,
        out_type=[
            jax.ShapeDtypeStruct((_E, _H), _F32),
            jax.ShapeDtypeStruct((_E, _H), _F32),
        ],
        mesh=_vector_mesh,
        scratch_types=[
            pltpu.VMEM((_NROW, _SUB), jnp.int32),
            pltpu.VMEM((_NROW, _SUB), jnp.int32),
            pltpu.VMEM((_GRP * _SUB, _H), _F32),
            pltpu.VMEM((_GRP * _SUB, _H), _F32),
            pltpu.VMEM((_GRP * _SUB, _H), _F32),
            pltpu.VMEM((_GRP * _SUB, _H), _F32),
            pltpu.SemaphoreType.DMA,
            pltpu.SemaphoreType.DMA,
            pltpu.SemaphoreType.DMA,
        ],
        compiler_params=_SC_PARAMS,
    )
    return k(xa, xb, src3, dst3)


def _mlp_body(ga_ref, gb_ref, z_ref, ss_ref, b1_ref, w2_ref, b2_ref,
              w3_ref, b3_ref, h3_ref):
    term = jnp.zeros(h3_ref.shape, _F32)
    for k in range(4):
        term = term + jnp.dot(z_ref[k].astype(jnp.bfloat16), ss_ref[k],
                              preferred_element_type=_F32)
    g = jnp.maximum(ga_ref[...] + gb_ref[...] + term + b1_ref[...], 0.0)
    outs = []
    for u in range(g.shape[1] // 128):
        s = g[:, u * 128:(u + 1) * 128]
        h = jnp.maximum(_dot(s, w2_ref[...]) + b2_ref[...], 0.0)
        h = jnp.maximum(_dot(h, w3_ref[...]) + b3_ref[...], 0.0)
        outs.append(h)
    h3_ref[...] = jnp.concatenate(outs, axis=1)


def _mlp(ga16, gb16, zz, ss, b1t, w2_blk, b2t, w3_blk, b3t):
    rows = _E // 128
    lb = 256
    full = lambda shape: pl.BlockSpec(shape, lambda *i: (0,) * len(shape))
    return pl.pallas_call(
        _mlp_body,
        grid=(2048 // lb,),
        in_specs=[
            pl.BlockSpec((rows, lb), lambda i: (0, i)),
            pl.BlockSpec((rows, lb), lambda i: (0, i)),
            pl.BlockSpec((4, rows, 128), lambda i: (0, 0, 0)),
            pl.BlockSpec((4, 128, lb), lambda i: (0, 0, i)),
            pl.BlockSpec((1, lb), lambda i: (0, i)),
            full((128, 128)),
            full((1, 128)),
            full((128, 128)),
            full((1, 128)),
        ],
        out_specs=pl.BlockSpec((rows, lb), lambda i: (0, i)),
        out_shape=jax.ShapeDtypeStruct((rows, 2048), _F32),
    )(ga16, gb16, zz, ss, b1t, w2_blk, b2t, w3_blk, b3t)


def _sc_scatter_body(h3_hbm, dsti_hbm, zero_hbm, s_hbm, cnt_hbm,
                     dstv, h0, h1, cntv, s_sh,
                     lsem0, lsem1, asem0, asem1):
    cid = lax.axis_index("c")
    sid = lax.axis_index("s")
    wid = sid * _NC + cid
    z16 = jnp.zeros((16,), _F32)
    ones16 = jnp.ones((16,), _F32)

    @pl.when(sid == 0)
    def _():
        pltpu.sync_copy(zero_hbm, s_sh)

    pltpu.sync_copy(dsti_hbm.at[wid], dstv)

    @pl.loop(0, _N, step=64)
    def _(i):
        for u in range(4):
            cntv[pl.ds(i + u * 16, 16)] = z16

    plsc.subcore_barrier()

    def fire_add(j0, h, asem):
        ds = []
        for k in range(_GRP):
            sl = pl.ds(k * _SUB, _SUB)
            ds.append(pltpu.async_copy(
                h.at[sl], s_sh.at[dstv.at[j0 + k]], asem, add=True))
        return ds

    @pl.loop(0, _NROW - _GRP, step=2 * _GRP)
    def _(j):
        base0 = wid * _EPW + j * _SUB
        l0 = pltpu.async_copy(
            h3_hbm.at[pl.ds(base0, _GRP * _SUB)], h0, lsem0)
        l1 = pltpu.async_copy(
            h3_hbm.at[pl.ds(base0 + _GRP * _SUB, _GRP * _SUB)], h1, lsem1)
        l0.wait()
        a0 = fire_add(j, h0, asem0)
        l1.wait()
        a1 = fire_add(j + _GRP, h1, asem1)
        for d in a0 + a1:
            d.wait()

    jt = _NROW - _GRP
    pltpu.sync_copy(h3_hbm.at[pl.ds(wid * _EPW + jt * _SUB, _GRP * _SUB)], h0)
    for d in fire_add(jt, h0, asem0):
        d.wait()

    @pl.loop(0, _NROW)
    def _(j):
        for u in range(_SUB // 16):
            plsc.addupdate_scatter(cntv, [dstv[j, pl.ds(u * 16, 16)]], ones16)

    plsc.subcore_barrier()

    @pl.when(sid == 0)
    def _():
        pltpu.sync_copy(s_sh, s_hbm.at[cid])

    pltpu.sync_copy(cntv, cnt_hbm.at[wid])


def _sc_scatter(h3, dst3, zeros):
    k = pl.kernel(
        _sc_scatter_body,
        out_type=[
            jax.ShapeDtypeStruct((_NC, _N, _H), _F32),
            jax.ShapeDtypeStruct((_NW, _N), _F32),
        ],
        mesh=_vector_mesh,
        scratch_types=[
            pltpu.VMEM((_NROW, _SUB), jnp.int32),
            pltpu.VMEM((_GRP * _SUB, _H), _F32),
            pltpu.VMEM((_GRP * _SUB, _H), _F32),
            pltpu.VMEM((_N,), _F32),
            pltpu.VMEM_SHARED((_N, _H), _F32),
            pltpu.SemaphoreType.DMA,
            pltpu.SemaphoreType.DMA,
            pltpu.SemaphoreType.DMA,
            pltpu.SemaphoreType.DMA,
        ],
        compiler_params=_SC_PARAMS,
    )
    return k(h3, dst3, zeros)


def _out_body(s_ref, c_ref, w4_ref, b4_ref, o_ref):
    s = s_ref[0] + s_ref[1]
    c = jnp.sum(c_ref[...].reshape(_NW, _N // 16, 16), axis=0)
    o_ref[...] = _dot(s, w4_ref[...]) + _dot(c, b4_ref[...])


def _out_gemm(s625, c3, w4_blk, b4_blk):
    return pl.pallas_call(
        _out_body,
        out_shape=jax.ShapeDtypeStruct((_N // 16, 16 * _D), _F32),
    )(s625, c3, w4_blk, b4_blk)


def kernel(x, edge_index, edge_attr, W1, b1, W2, b2, W3, b3, W4, b4):
    w1a = W1[:_D]
    w1b = W1[_D:2 * _D]
    w1c = W1[2 * _D:]

    eye8 = jnp.eye(8, dtype=_F32)
    wa_blk = jnp.kron(eye8, w1a)
    wb_blk = jnp.kron(eye8, w1b)
    eye128 = jnp.eye(128, dtype=_F32)
    ss = jnp.stack([jnp.kron(eye128, w1c[k:k + 1]) for k in range(4)])
    ss = ss.astype(jnp.bfloat16)
    w2_blk = jnp.kron(eye8, W2)
    w3_blk = jnp.kron(eye8, W3)
    b1t = jnp.tile(b1, 128).reshape(1, 2048)
    b2t = jnp.tile(b2, 8).reshape(1, 128)
    b3t = jnp.tile(b3, 8).reshape(1, 128)
    eye16 = jnp.eye(16, dtype=_F32)
    w4_blk = jnp.kron(eye16, W4)
    b4_blk = jnp.kron(eye16, b4.reshape(1, _D))

    src3 = edge_index[0].reshape(_NW, _NROW, _SUB)
    dst3 = edge_index[1].reshape(_NW, _NROW, _SUB)

    xa8, xb8 = _node_proj(x.reshape(_N // 8, 8 * _D), wa_blk, wb_blk)
    ga, gb = _sc_gather(
        xa8.reshape(_N, _H), xb8.reshape(_N, _H), src3, dst3)

    ea_t = jax.lax.optimization_barrier(edge_attr.T)
    zz = ea_t.reshape(4, _E // 128, 128)

    h3 = _mlp(
        ga.reshape(_E // 128, 2048),
        gb.reshape(_E // 128, 2048),
        zz, ss, b1t, w2_blk, b2t, w3_blk, b3t,
    )

    zeros = jnp.zeros((_N, _H), _F32)
    s_part, cnt = _sc_scatter(h3.reshape(_E, _H), dst3, zeros)

    out = _out_gemm(
        s_part.reshape(_NC, _N // 16, 16 * _H),
        cnt.reshape(_NW * (_N // 16), 16),
        w4_blk, b4_blk,
    )
    return out.reshape(_N, _D)
